# baseline (speedup 1.0000x reference)
import functools

import jax
import jax.numpy as jnp
import numpy as np
from jax import lax
from jax.experimental import pallas as pl
from jax.experimental.pallas import tpu as pltpu
from jax.experimental.pallas import tpu_sc as plsc

_VOCAB = 1_000_000
_D = 16
_N_TOK = 16384
_N_NEG = 2048

_KS1 = 42
_KS2 = (0x1BD11BDA ^ 42) & 0xFFFFFFFF

_CR = 192
_VR = -(-_VOCAB // 128)
_VRP = -(-_VR // _CR) * _CR
_VP = _VRP * 128


def _rotl(x, d):
    return (x << d) | (x >> (32 - d))


def _bits_from_x1(b):
    b = b + jnp.uint32(_KS1)
    a = b
    b = _rotl(b, 13) ^ a
    for r in (15, 26, 6):
        a = a + b
        b = _rotl(b, r) ^ a
    a = a + jnp.uint32(_KS1)
    b = b + jnp.uint32(_KS2 + 1)
    for r in (17, 29, 16, 24):
        a = a + b
        b = _rotl(b, r) ^ a
    a = a + jnp.uint32(_KS2)
    b = b + jnp.uint32(2)
    for r in (13, 15, 26, 6):
        a = a + b
        b = _rotl(b, r) ^ a
    b = b + jnp.uint32(_KS1 + 3)
    for r in (17, 29, 16, 24):
        a = a + b
        b = _rotl(b, r) ^ a
    a = a + jnp.uint32(_KS1)
    b = b + jnp.uint32(_KS2 + 4)
    for r in (13, 15, 26, 6):
        a = a + b
        b = _rotl(b, r) ^ a
    a = a + jnp.uint32(_KS2)
    b = b + jnp.uint32(5)
    return a ^ b


def _recip_body(p_ref, o_ref):
    o_ref[...] = jnp.float32(2.0 ** -23) / (p_ref[...] + jnp.float32(1e-10))


def _recip(p_pad, interpret=False):
    return pl.pallas_call(
        _recip_body,
        out_shape=jax.ShapeDtypeStruct(p_pad.shape, jnp.float32),
        interpret=interpret,
    )(p_pad)


def _sample_body(r_ref, out_ref, *, vocab, n_chunk, cr):
    s = pl.program_id(0)
    base = (s * vocab).astype(jnp.uint32)
    voff = (lax.broadcasted_iota(jnp.uint32, (cr, 128), 0) * 128
            + lax.broadcasted_iota(jnp.uint32, (cr, 128), 1))

    def chunk(c, carry):
        minv, mini = carry
        x1 = voff + (base + (c * (cr * 128)).astype(jnp.uint32))
        bits = _bits_from_x1(x1)
        w = (jnp.uint32(1 << 23) - (bits >> 9)).astype(jnp.float32)
        score = w * r_ref[pl.ds(c * cr, cr), :]
        pred = score < minv
        minv = jnp.where(pred, score, minv)
        mini = jnp.where(pred, x1, mini)
        return minv, mini

    minv, mini = lax.fori_loop(
        0, n_chunk, chunk,
        (jnp.full((cr, 128), jnp.inf, jnp.float32),
         jnp.zeros((cr, 128), jnp.uint32)))
    mall = jnp.min(minv)
    mflip = lax.bitcast_convert_type(mini ^ jnp.uint32(0x80000000), jnp.int32)
    bestf = jnp.min(jnp.where(minv == mall, mflip, jnp.int32(2 ** 31 - 1)))
    best = (lax.bitcast_convert_type(bestf, jnp.uint32) ^ jnp.uint32(0x80000000)
            ) - base
    out_ref[...] = jnp.broadcast_to(best.astype(jnp.int32), (1, 1, 1))


def _sample(r_pad, n_samples, vocab, vrp, cr, interpret=False):
    n_chunk = vrp // cr
    return pl.pallas_call(
        functools.partial(_sample_body, vocab=vocab, n_chunk=n_chunk, cr=cr),
        grid=(n_samples,),
        in_specs=[pl.BlockSpec((vrp, 128), lambda s: (0, 0))],
        out_specs=pl.BlockSpec((1, 1, 1), lambda s: (s, 0, 0)),
        out_shape=jax.ShapeDtypeStruct((n_samples, 1, 1), jnp.int32),
        interpret=interpret,
    )(r_pad)


def _gather(emb, probs, idx, batch):
    nw = 32
    bpw = batch // nw
    mesh = plsc.VectorSubcoreMesh(core_axis_name="c", subcore_axis_name="s")

    @functools.partial(
        pl.kernel, mesh=mesh,
        compiler_params=pltpu.CompilerParams(use_tc_tiling_on_sc=False),
        out_type=[jax.ShapeDtypeStruct((batch, _D), jnp.float32),
                  jax.ShapeDtypeStruct((batch,), jnp.float32)],
        scratch_types=[
            pltpu.VMEM((bpw,), jnp.int32),
            pltpu.VMEM((bpw, _D), jnp.float32),
            pltpu.VMEM((bpw,), jnp.float32),
            pltpu.SemaphoreType.DMA,
            pltpu.SemaphoreType.DMA,
        ],
    )
    def k(emb_hbm, probs_hbm, idx_hbm, rows_out, p_out, idx_v, rows_v, p_v,
          sem1, sem2):
        wid = lax.axis_index("s") * 2 + lax.axis_index("c")
        base = wid * bpw
        pltpu.sync_copy(idx_hbm.at[pl.ds(base, bpw)], idx_v)
        c1 = pltpu.async_copy(emb_hbm.at[idx_v], rows_v, sem1)
        c2 = pltpu.async_copy(probs_hbm.at[idx_v], p_v, sem2)
        c1.wait()
        c2.wait()
        pltpu.sync_copy(rows_v, rows_out.at[pl.ds(base, bpw)])
        pltpu.sync_copy(p_v, p_out.at[pl.ds(base, bpw)])

    return k(emb, probs, idx)


def _loss_body(h_ref, ep_ref, y_ref, tp_ref, en_ref, sid_ref, sp_ref,
               out_ref, acc_ref):
    i = pl.program_id(0)

    @pl.when(i == 0)
    def _init():
        acc_ref[0] = jnp.float32(0.0)
        acc_ref[1] = jnp.float32(0.0)

    h = h_ref[...]
    y = y_ref[0]
    tp = tp_ref[0]
    en = en_ref[...]
    sid = sid_ref[...]
    sp = sp_ref[...]

    neg = lax.dot_general(h, en, (((1,), (1,)), ((), ())),
                          preferred_element_type=jnp.float32)
    neg = jnp.where(y == sid, jnp.float32(-1e9), neg)
    negl = neg - jnp.log(sp + jnp.float32(1e-10))
    pos = (jnp.sum(h * ep_ref[...], axis=1, keepdims=True)
           - jnp.log(tp + jnp.float32(1e-10)))
    mx = jnp.maximum(jnp.max(negl, axis=1, keepdims=True), pos)
    se = jnp.sum(jnp.exp(negl - mx), axis=1, keepdims=True) + jnp.exp(pos - mx)
    pt = mx + jnp.log(se) - pos
    msk = y != 0
    acc_ref[0] += jnp.sum(jnp.where(msk, pt, jnp.float32(0.0)))
    acc_ref[1] += jnp.sum(msk.astype(jnp.float32))

    @pl.when(i == pl.num_programs(0) - 1)
    def _fin():
        out_ref[0, 0] = acc_ref[0] / acc_ref[1]


def _loss(hidden, e_pos, y3, tp3, e_neg, sid2, sp2, n_tok, n_neg,
          interpret=False):
    tb = 2048 if n_tok % 2048 == 0 else n_tok
    grid = n_tok // tb
    return pl.pallas_call(
        _loss_body,
        grid=(grid,),
        in_specs=[
            pl.BlockSpec((tb, _D), lambda i: (i, 0)),
            pl.BlockSpec((tb, _D), lambda i: (i, 0)),
            pl.BlockSpec((1, tb, 1), lambda i: (i, 0, 0)),
            pl.BlockSpec((1, tb, 1), lambda i: (i, 0, 0)),
            pl.BlockSpec((n_neg, _D), lambda i: (0, 0)),
            pl.BlockSpec((1, n_neg), lambda i: (0, 0)),
            pl.BlockSpec((1, n_neg), lambda i: (0, 0)),
        ],
        out_specs=pl.BlockSpec((1, 1), lambda i: (0, 0), memory_space=pltpu.SMEM),
        out_shape=jax.ShapeDtypeStruct((1, 1), jnp.float32),
        scratch_shapes=[pltpu.SMEM((2,), jnp.float32)],
        interpret=interpret,
    )(hidden, e_pos, y3, tp3, e_neg, sid2, sp2)


def kernel(hidden, y, emb_table, sampling_probs):
    y = y.astype(jnp.int32)
    p_pad = jnp.pad(sampling_probs, (0, _VP - _VOCAB)).reshape(_VRP, 128)
    r_pad = _recip(p_pad)
    sampled = _sample(r_pad, _N_NEG, _VOCAB, _VRP, _CR).reshape(_N_NEG)
    e_pos, tp = _gather(emb_table, sampling_probs, y, _N_TOK)
    e_neg, sp = _gather(emb_table, sampling_probs, sampled, _N_NEG)
    loss = _loss(hidden, e_pos,
                 y.reshape(_N_TOK // 2048, 2048, 1),
                 tp.reshape(_N_TOK // 2048, 2048, 1),
                 e_neg,
                 sampled.reshape(1, _N_NEG),
                 sp.reshape(1, _N_NEG),
                 _N_TOK, _N_NEG)
    return loss[0, 0]


# speedup vs baseline: 1.0338x; 1.0338x over previous
import functools

import jax
import jax.numpy as jnp
import numpy as np
from jax import lax
from jax.experimental import pallas as pl
from jax.experimental.pallas import tpu as pltpu
from jax.experimental.pallas import tpu_sc as plsc

_VOCAB = 1_000_000
_D = 16
_N_TOK = 16384
_N_NEG = 2048

_KS1 = 42
_KS2 = (0x1BD11BDA ^ 42) & 0xFFFFFFFF

_CR = 96
_VR = -(-_VOCAB // 128)
_VRP = -(-_VR // _CR) * _CR
_VP = _VRP * 128


def _rotl(x, d):
    return (x << d) | (x >> (32 - d))


def _bits_from_x1(b):
    b = b + jnp.uint32(_KS1)
    a = b
    b = _rotl(b, 13) ^ a
    for r in (15, 26, 6):
        a = a + b
        b = _rotl(b, r) ^ a
    a = a + jnp.uint32(_KS1)
    b = b + jnp.uint32(_KS2 + 1)
    for r in (17, 29, 16, 24):
        a = a + b
        b = _rotl(b, r) ^ a
    a = a + jnp.uint32(_KS2)
    b = b + jnp.uint32(2)
    for r in (13, 15, 26, 6):
        a = a + b
        b = _rotl(b, r) ^ a
    b = b + jnp.uint32(_KS1 + 3)
    for r in (17, 29, 16, 24):
        a = a + b
        b = _rotl(b, r) ^ a
    a = a + jnp.uint32(_KS1)
    b = b + jnp.uint32(_KS2 + 4)
    for r in (13, 15, 26, 6):
        a = a + b
        b = _rotl(b, r) ^ a
    a = a + jnp.uint32(_KS2)
    b = b + jnp.uint32(5)
    return a ^ b


def _recip_body(p_ref, o_ref):
    o_ref[...] = jnp.float32(2.0 ** -23) / (p_ref[...] + jnp.float32(1e-10))


def _recip(p_pad, interpret=False):
    return pl.pallas_call(
        _recip_body,
        out_shape=jax.ShapeDtypeStruct(p_pad.shape, jnp.float32),
        interpret=interpret,
    )(p_pad)


def _sample_body(r_ref, out_ref, *, vocab, n_chunk, cr):
    s = pl.program_id(0)
    base = (s * vocab).astype(jnp.uint32)
    voff = (lax.broadcasted_iota(jnp.uint32, (cr, 128), 0) * 128
            + lax.broadcasted_iota(jnp.uint32, (cr, 128), 1))

    def chunk(c, carry):
        minv, mini = carry
        x1 = voff + (base + (c * (cr * 128)).astype(jnp.uint32))
        bits = _bits_from_x1(x1)
        w = (jnp.uint32(1 << 23) - (bits >> 9)).astype(jnp.float32)
        score = w * r_ref[pl.ds(c * cr, cr), :]
        pred = score < minv
        minv = jnp.where(pred, score, minv)
        mini = jnp.where(pred, x1, mini)
        return minv, mini

    minv, mini = lax.fori_loop(
        0, n_chunk, chunk,
        (jnp.full((cr, 128), jnp.inf, jnp.float32),
         jnp.zeros((cr, 128), jnp.uint32)))
    mall = jnp.min(minv)
    mflip = lax.bitcast_convert_type(mini ^ jnp.uint32(0x80000000), jnp.int32)
    bestf = jnp.min(jnp.where(minv == mall, mflip, jnp.int32(2 ** 31 - 1)))
    best = (lax.bitcast_convert_type(bestf, jnp.uint32) ^ jnp.uint32(0x80000000)
            ) - base
    out_ref[...] = jnp.broadcast_to(best.astype(jnp.int32), (1, 1, 1))


def _sample(r_pad, n_samples, vocab, vrp, cr, interpret=False):
    n_chunk = vrp // cr
    return pl.pallas_call(
        functools.partial(_sample_body, vocab=vocab, n_chunk=n_chunk, cr=cr),
        grid=(n_samples,),
        in_specs=[pl.BlockSpec((vrp, 128), lambda s: (0, 0))],
        out_specs=pl.BlockSpec((1, 1, 1), lambda s: (s, 0, 0)),
        out_shape=jax.ShapeDtypeStruct((n_samples, 1, 1), jnp.int32),
        interpret=interpret,
    )(r_pad)


def _gather(emb, probs, idx, batch):
    nw = 32
    bpw = batch // nw
    mesh = plsc.VectorSubcoreMesh(core_axis_name="c", subcore_axis_name="s")

    @functools.partial(
        pl.kernel, mesh=mesh,
        compiler_params=pltpu.CompilerParams(use_tc_tiling_on_sc=False),
        out_type=[jax.ShapeDtypeStruct((batch, _D), jnp.float32),
                  jax.ShapeDtypeStruct((batch,), jnp.float32)],
        scratch_types=[
            pltpu.VMEM((bpw,), jnp.int32),
            pltpu.VMEM((bpw, _D), jnp.float32),
            pltpu.VMEM((bpw,), jnp.float32),
            pltpu.SemaphoreType.DMA,
            pltpu.SemaphoreType.DMA,
        ],
    )
    def k(emb_hbm, probs_hbm, idx_hbm, rows_out, p_out, idx_v, rows_v, p_v,
          sem1, sem2):
        wid = lax.axis_index("s") * 2 + lax.axis_index("c")
        base = wid * bpw
        pltpu.sync_copy(idx_hbm.at[pl.ds(base, bpw)], idx_v)
        c1 = pltpu.async_copy(emb_hbm.at[idx_v], rows_v, sem1)
        c2 = pltpu.async_copy(probs_hbm.at[idx_v], p_v, sem2)
        c1.wait()
        c2.wait()
        pltpu.sync_copy(rows_v, rows_out.at[pl.ds(base, bpw)])
        pltpu.sync_copy(p_v, p_out.at[pl.ds(base, bpw)])

    return k(emb, probs, idx)


def _loss_body(h_ref, ep_ref, y_ref, tp_ref, en_ref, sid_ref, sp_ref,
               out_ref, acc_ref):
    i = pl.program_id(0)

    @pl.when(i == 0)
    def _init():
        acc_ref[0] = jnp.float32(0.0)
        acc_ref[1] = jnp.float32(0.0)

    h = h_ref[...]
    y = y_ref[0]
    tp = tp_ref[0]
    en = en_ref[...]
    sid = sid_ref[...]
    sp = sp_ref[...]

    neg = lax.dot_general(h, en, (((1,), (1,)), ((), ())),
                          preferred_element_type=jnp.float32)
    neg = jnp.where(y == sid, jnp.float32(-1e9), neg)
    negl = neg - jnp.log(sp + jnp.float32(1e-10))
    pos = (jnp.sum(h * ep_ref[...], axis=1, keepdims=True)
           - jnp.log(tp + jnp.float32(1e-10)))
    mx = jnp.maximum(jnp.max(negl, axis=1, keepdims=True), pos)
    se = jnp.sum(jnp.exp(negl - mx), axis=1, keepdims=True) + jnp.exp(pos - mx)
    pt = mx + jnp.log(se) - pos
    msk = y != 0
    acc_ref[0] += jnp.sum(jnp.where(msk, pt, jnp.float32(0.0)))
    acc_ref[1] += jnp.sum(msk.astype(jnp.float32))

    @pl.when(i == pl.num_programs(0) - 1)
    def _fin():
        out_ref[0, 0] = acc_ref[0] / acc_ref[1]


def _loss(hidden, e_pos, y3, tp3, e_neg, sid2, sp2, n_tok, n_neg,
          interpret=False):
    tb = 2048 if n_tok % 2048 == 0 else n_tok
    grid = n_tok // tb
    return pl.pallas_call(
        _loss_body,
        grid=(grid,),
        in_specs=[
            pl.BlockSpec((tb, _D), lambda i: (i, 0)),
            pl.BlockSpec((tb, _D), lambda i: (i, 0)),
            pl.BlockSpec((1, tb, 1), lambda i: (i, 0, 0)),
            pl.BlockSpec((1, tb, 1), lambda i: (i, 0, 0)),
            pl.BlockSpec((n_neg, _D), lambda i: (0, 0)),
            pl.BlockSpec((1, n_neg), lambda i: (0, 0)),
            pl.BlockSpec((1, n_neg), lambda i: (0, 0)),
        ],
        out_specs=pl.BlockSpec((1, 1), lambda i: (0, 0), memory_space=pltpu.SMEM),
        out_shape=jax.ShapeDtypeStruct((1, 1), jnp.float32),
        scratch_shapes=[pltpu.SMEM((2,), jnp.float32)],
        interpret=interpret,
    )(hidden, e_pos, y3, tp3, e_neg, sid2, sp2)


def kernel(hidden, y, emb_table, sampling_probs):
    y = y.astype(jnp.int32)
    p_pad = jnp.pad(sampling_probs, (0, _VP - _VOCAB)).reshape(_VRP, 128)
    r_pad = _recip(p_pad)
    sampled = _sample(r_pad, _N_NEG, _VOCAB, _VRP, _CR).reshape(_N_NEG)
    e_pos, tp = _gather(emb_table, sampling_probs, y, _N_TOK)
    e_neg, sp = _gather(emb_table, sampling_probs, sampled, _N_NEG)
    loss = _loss(hidden, e_pos,
                 y.reshape(_N_TOK // 2048, 2048, 1),
                 tp.reshape(_N_TOK // 2048, 2048, 1),
                 e_neg,
                 sampled.reshape(1, _N_NEG),
                 sp.reshape(1, _N_NEG),
                 _N_TOK, _N_NEG)
    return loss[0, 0]
